# baseline (speedup 1.0000x reference)
import functools

import jax
import jax.numpy as jnp
from jax import lax
from jax.experimental import pallas as pl
from jax.experimental.pallas import tpu as pltpu
from jax.experimental.pallas import tpu_sc as plsc

B = 16384
F = 64
C = 128

_NC = 2
_NS = 16
_NW = _NC * _NS
_ROWS_PER_W = B // _NW
_CHUNK = 128
_NCHUNK = _ROWS_PER_W // _CHUNK


def _tc_body(cnn_ref, e_ref, bp_ref, alpha_ref, proj_ref, dense_ref):
    cnn = cnn_ref[...]
    proj_ref[...] = lax.dot_general(
        cnn, e_ref[...], (((1,), (1,)), ((), ())),
        preferred_element_type=jnp.float32)
    dense_ref[...] = jnp.sum(cnn * bp_ref[...], axis=1,
                             keepdims=True) + alpha_ref[0, 0]


def _tc_dense(cnn, E_w, bp_w, alpha):
    return pl.pallas_call(
        _tc_body,
        out_shape=(
            jax.ShapeDtypeStruct((B, F), jnp.float32),
            jax.ShapeDtypeStruct((B, 1), jnp.float32),
        ),
        in_specs=[
            pl.BlockSpec(memory_space=pltpu.MemorySpace.VMEM),
            pl.BlockSpec(memory_space=pltpu.MemorySpace.VMEM),
            pl.BlockSpec(memory_space=pltpu.MemorySpace.VMEM),
            pl.BlockSpec(memory_space=pltpu.MemorySpace.SMEM),
        ],
        out_specs=(
            pl.BlockSpec(memory_space=pltpu.MemorySpace.VMEM),
            pl.BlockSpec(memory_space=pltpu.MemorySpace.VMEM),
        ),
    )(cnn, E_w, bp_w, alpha)


def _sc_body(user_h, item_h, proj_h, dense_h, bu_h, bi_h, gu_h, gi_h, tu_h,
             out_h, idx_u, idx_i, gu, gi, tu, pj, bu, bi, dns, ob, sem):
    wid = lax.axis_index("s") * _NC + lax.axis_index("c")
    base_w = wid * _ROWS_PER_W
    riota = lax.iota(jnp.int32, 16)
    z16 = jnp.zeros((16,), jnp.int32)

    for ch in range(_NCHUNK):
        base = pl.multiple_of(base_w + ch * _CHUNK, _CHUNK)
        pltpu.sync_copy(user_h.at[pl.ds(base, _CHUNK)], idx_u)
        pltpu.sync_copy(item_h.at[pl.ds(base, _CHUNK)], idx_i)
        cps = (
            pltpu.async_copy(gu_h.at[idx_u], gu, sem),
            pltpu.async_copy(gi_h.at[idx_i], gi, sem),
            pltpu.async_copy(tu_h.at[idx_u], tu, sem),
            pltpu.async_copy(proj_h.at[pl.ds(base, _CHUNK), :], pj, sem),
            pltpu.async_copy(bu_h.at[idx_u], bu, sem),
            pltpu.async_copy(bi_h.at[idx_i], bi, sem),
            pltpu.async_copy(dense_h.at[pl.ds(base, _CHUNK)], dns, sem),
        )
        for cp in cps:
            cp.wait()

        def group(g, carry):
            r0 = pl.multiple_of(g * 16, 16)
            ridx = riota + g * 16
            acc = dns[pl.ds(r0, 16)]
            acc = acc + bu[pl.ds(r0, 16)]
            acc = acc + bi[pl.ds(r0, 16)]
            for f in range(F):
                cf = jnp.full((16,), f, jnp.int32)
                acc = acc + plsc.load_gather(gu, [ridx, cf]) * plsc.load_gather(gi, [ridx, cf])
                acc = acc + plsc.load_gather(tu, [ridx, cf]) * plsc.load_gather(pj, [ridx, cf])
            ob[pl.ds(r0, 16)] = acc
            return carry

        lax.fori_loop(0, _CHUNK // 16, group, 0)
        pltpu.sync_copy(ob, out_h.at[pl.ds(base, _CHUNK)])


def _sc_combine(user, item, proj, dense, bu_w, bi_w, gu_w, gi_w, tu_w):
    mesh = plsc.VectorSubcoreMesh(core_axis_name="c", subcore_axis_name="s")
    return pl.kernel(
        _sc_body,
        out_type=jax.ShapeDtypeStruct((B,), jnp.float32),
        mesh=mesh,
        compiler_params=pltpu.CompilerParams(
            needs_layout_passes=False, use_tc_tiling_on_sc=False),
        scratch_types=[
            pltpu.VMEM((_CHUNK,), jnp.int32),
            pltpu.VMEM((_CHUNK,), jnp.int32),
            pltpu.VMEM((_CHUNK, F), jnp.float32),
            pltpu.VMEM((_CHUNK, F), jnp.float32),
            pltpu.VMEM((_CHUNK, F), jnp.float32),
            pltpu.VMEM((_CHUNK, F), jnp.float32),
            pltpu.VMEM((_CHUNK,), jnp.float32),
            pltpu.VMEM((_CHUNK,), jnp.float32),
            pltpu.VMEM((_CHUNK,), jnp.float32),
            pltpu.VMEM((_CHUNK,), jnp.float32),
            pltpu.SemaphoreType.DMA,
        ],
    )(user, item, proj, dense, bu_w, bi_w, gu_w, gi_w, tu_w)


def kernel(user, item_i, cnn_feature_i, alpha, beta_u_w, beta_i_w,
           gamma_u_w, gamma_i_w, theta_u_w, E_w, beta_p_w):
    user = user.astype(jnp.int32)
    item = item_i.astype(jnp.int32)
    proj, dense = _tc_dense(cnn_feature_i, E_w, beta_p_w, alpha)
    out = _sc_combine(user, item, proj, dense.reshape(B),
                      beta_u_w.reshape(-1), beta_i_w.reshape(-1),
                      gamma_u_w, gamma_i_w, theta_u_w)
    return out.reshape(1, B)


# speedup vs baseline: 1.0701x; 1.0701x over previous
import jax
import jax.numpy as jnp
from jax import lax
from jax.experimental import pallas as pl
from jax.experimental.pallas import tpu as pltpu
from jax.experimental.pallas import tpu_sc as plsc

B = 16384
F = 64
C = 128

_NC = 2
_NS = 16
_NW = _NC * _NS
_ROWS_PER_W = B // _NW
_CHUNK = 128
_NCHUNK = _ROWS_PER_W // _CHUNK


def _tc_body(cnn_ref, w_ref, alpha_ref, aux_ref):
    cnn = cnn_ref[...]
    aux = lax.dot_general(cnn, w_ref[...], (((1,), (1,)), ((), ())),
                          preferred_element_type=jnp.float32)
    is_dense_col = (lax.broadcasted_iota(jnp.int32, (1, C), 1) == F)
    aux_ref[...] = aux + jnp.where(is_dense_col, alpha_ref[0, 0], 0.0)


def _tc_dense(cnn, W, alpha):
    return pl.pallas_call(
        _tc_body,
        out_shape=jax.ShapeDtypeStruct((B, C), jnp.float32),
        in_specs=[
            pl.BlockSpec(memory_space=pltpu.MemorySpace.VMEM),
            pl.BlockSpec(memory_space=pltpu.MemorySpace.VMEM),
            pl.BlockSpec(memory_space=pltpu.MemorySpace.SMEM),
        ],
        out_specs=pl.BlockSpec(memory_space=pltpu.MemorySpace.VMEM),
    )(cnn, W, alpha)


def _sc_body(user_h, item_h, aux_h, bu_h, bi_h, gu_h, gi_h, tu_h,
             out_h, idx_u, idx_i, idx_uh, idx_ih, gu, gi, tu, ax, bu, bi,
             ob, sem):
    wid = lax.axis_index("s") * _NC + lax.axis_index("c")
    base_w = wid * _ROWS_PER_W
    riota = lax.iota(jnp.int32, 16)

    for ch in range(_NCHUNK):
        base = pl.multiple_of(base_w + ch * _CHUNK, _CHUNK)
        pltpu.sync_copy(user_h.at[pl.ds(base, _CHUNK)], idx_u)
        pltpu.sync_copy(item_h.at[pl.ds(base, _CHUNK)], idx_i)
        for j in range(_CHUNK // 16):
            idx_uh[pl.ds(j * 16, 16)] = idx_u[pl.ds(j * 16, 16)] >> 1
            idx_ih[pl.ds(j * 16, 16)] = idx_i[pl.ds(j * 16, 16)] >> 1
        cps = (
            pltpu.async_copy(gu_h.at[idx_uh], gu, sem),
            pltpu.async_copy(gi_h.at[idx_ih], gi, sem),
            pltpu.async_copy(tu_h.at[idx_uh], tu, sem),
            pltpu.async_copy(aux_h.at[pl.ds(base, _CHUNK), :], ax, sem),
            pltpu.async_copy(bu_h.at[idx_u], bu, sem),
            pltpu.async_copy(bi_h.at[idx_i], bi, sem),
        )
        for cp in cps:
            cp.wait()

        def group(g, carry):
            r0 = pl.multiple_of(g * 16, 16)
            ridx = riota + g * 16
            cbu = (idx_u[pl.ds(r0, 16)] & 1) << 6
            cbi = (idx_i[pl.ds(r0, 16)] & 1) << 6
            a0 = bu[pl.ds(r0, 16)] + bi[pl.ds(r0, 16)]
            a0 = a0 + plsc.load_gather(ax, [ridx, jnp.full((16,), F, jnp.int32)])
            a1 = jnp.zeros((16,), jnp.float32)
            a2 = jnp.zeros((16,), jnp.float32)
            a3 = jnp.zeros((16,), jnp.float32)
            for f in range(F):
                cf = jnp.full((16,), f, jnp.int32)
                g_prod = plsc.load_gather(gu, [ridx, cbu + f]) * plsc.load_gather(gi, [ridx, cbi + f])
                t_prod = plsc.load_gather(tu, [ridx, cbu + f]) * plsc.load_gather(ax, [ridx, cf])
                if f % 2 == 0:
                    a0 = a0 + g_prod
                    a2 = a2 + t_prod
                else:
                    a1 = a1 + g_prod
                    a3 = a3 + t_prod
            ob[pl.ds(r0, 16)] = (a0 + a1) + (a2 + a3)
            return carry

        lax.fori_loop(0, _CHUNK // 16, group, 0)
        pltpu.sync_copy(ob, out_h.at[pl.ds(base, _CHUNK)])


def _sc_combine(user, item, aux, bu_w, bi_w, gu2, gi2, tu2):
    mesh = plsc.VectorSubcoreMesh(core_axis_name="c", subcore_axis_name="s")
    return pl.kernel(
        _sc_body,
        out_type=jax.ShapeDtypeStruct((B,), jnp.float32),
        mesh=mesh,
        compiler_params=pltpu.CompilerParams(
            needs_layout_passes=False, use_tc_tiling_on_sc=False),
        scratch_types=[
            pltpu.VMEM((_CHUNK,), jnp.int32),
            pltpu.VMEM((_CHUNK,), jnp.int32),
            pltpu.VMEM((_CHUNK,), jnp.int32),
            pltpu.VMEM((_CHUNK,), jnp.int32),
            pltpu.VMEM((_CHUNK, C), jnp.float32),
            pltpu.VMEM((_CHUNK, C), jnp.float32),
            pltpu.VMEM((_CHUNK, C), jnp.float32),
            pltpu.VMEM((_CHUNK, C), jnp.float32),
            pltpu.VMEM((_CHUNK,), jnp.float32),
            pltpu.VMEM((_CHUNK,), jnp.float32),
            pltpu.VMEM((_CHUNK,), jnp.float32),
            pltpu.SemaphoreType.DMA,
        ],
    )(user, item, aux, bu_w, bi_w, gu2, gi2, tu2)


def kernel(user, item_i, cnn_feature_i, alpha, beta_u_w, beta_i_w,
           gamma_u_w, gamma_i_w, theta_u_w, E_w, beta_p_w):
    user = user.astype(jnp.int32)
    item = item_i.astype(jnp.int32)
    W = jnp.concatenate(
        [E_w, beta_p_w, jnp.zeros((C - F - 1, C), jnp.float32)], axis=0)
    aux = _tc_dense(cnn_feature_i, W, alpha)
    U2 = gamma_u_w.shape[0] // 2
    I2 = gamma_i_w.shape[0] // 2
    out = _sc_combine(user, item, aux,
                      beta_u_w.reshape(-1), beta_i_w.reshape(-1),
                      gamma_u_w.reshape(U2, C), gamma_i_w.reshape(I2, C),
                      theta_u_w.reshape(U2, C))
    return out.reshape(1, B)


# speedup vs baseline: 1.2594x; 1.1770x over previous
import jax
import jax.numpy as jnp
from jax import lax
from jax.experimental import pallas as pl
from jax.experimental.pallas import tpu as pltpu
from jax.experimental.pallas import tpu_sc as plsc

B = 16384
F = 64
C = 128

_NC = 2
_NS = 16
_NW = _NC * _NS
_ROWS_PER_W = B // _NW
_CHUNK = 128
_NCHUNK = _ROWS_PER_W // _CHUNK


def _tc_body(cnn_ref, w_ref, alpha_ref, aux_ref):
    cnn = cnn_ref[...]
    aux = lax.dot_general(cnn, w_ref[...], (((1,), (1,)), ((), ())),
                          preferred_element_type=jnp.float32)
    is_dense_col = (lax.broadcasted_iota(jnp.int32, (1, C), 1) == F)
    aux_ref[...] = aux + jnp.where(is_dense_col, alpha_ref[0, 0], 0.0)


def _tc_dense(cnn, W, alpha):
    return pl.pallas_call(
        _tc_body,
        out_shape=jax.ShapeDtypeStruct((B, C), jnp.float32),
        in_specs=[
            pl.BlockSpec(memory_space=pltpu.MemorySpace.VMEM),
            pl.BlockSpec(memory_space=pltpu.MemorySpace.VMEM),
            pl.BlockSpec(memory_space=pltpu.MemorySpace.SMEM),
        ],
        out_specs=pl.BlockSpec(memory_space=pltpu.MemorySpace.VMEM),
    )(cnn, W, alpha)


def _sc_body(user_h, item_h, aux_h, bu_h, bi_h, cat_h, gi_h,
             out_h, idx_u2, idx_i2, idx_ih2,
             cu0, cu1, ci0, ci1, ax0, ax1, bu0, bu1, bi0, bi1, ob0, ob1,
             sem_idx, sem0, sem1):
    wid = lax.axis_index("s") * _NC + lax.axis_index("c")
    base_w = wid * _ROWS_PER_W
    riota = lax.iota(jnp.int32, 16)

    cu = (cu0, cu1)
    ci = (ci0, ci1)
    ax = (ax0, ax1)
    bu = (bu0, bu1)
    bi = (bi0, bi1)
    ob = (ob0, ob1)
    sems = (sem0, sem1)

    idx_cps = []
    for ch in range(_NCHUNK):
        base = pl.multiple_of(base_w + ch * _CHUNK, _CHUNK)
        idx_cps.append(
            pltpu.async_copy(user_h.at[pl.ds(base, _CHUNK)], idx_u2.at[ch],
                             sem_idx))
        idx_cps.append(
            pltpu.async_copy(item_h.at[pl.ds(base, _CHUNK)], idx_i2.at[ch],
                             sem_idx))
    for cp in idx_cps:
        cp.wait()
    for ch in range(_NCHUNK):
        for j in range(_CHUNK // 16):
            idx_ih2[ch, pl.ds(j * 16, 16)] = idx_i2[ch, pl.ds(j * 16, 16)] >> 1

    def issue(ch, s):
        base = pl.multiple_of(base_w + ch * _CHUNK, _CHUNK)
        return (
            pltpu.async_copy(cat_h.at[idx_u2.at[ch]], cu[s], sems[s]),
            pltpu.async_copy(gi_h.at[idx_ih2.at[ch]], ci[s], sems[s]),
            pltpu.async_copy(aux_h.at[pl.ds(base, _CHUNK), :], ax[s], sems[s]),
            pltpu.async_copy(bu_h.at[idx_u2.at[ch]], bu[s], sems[s]),
            pltpu.async_copy(bi_h.at[idx_i2.at[ch]], bi[s], sems[s]),
        )

    inflight = issue(0, 0)
    for ch in range(_NCHUNK):
        s = ch % 2
        nxt = None
        if ch + 1 < _NCHUNK:
            nxt = issue(ch + 1, (ch + 1) % 2)
        for cp in inflight:
            cp.wait()
        inflight = nxt

        def group(g, carry):
            r0 = pl.multiple_of(g * 16, 16)
            ridx = riota + g * 16
            cbi = (idx_i2[ch, pl.ds(r0, 16)] & 1) << 6
            a0 = bu[s][pl.ds(r0, 16)] + bi[s][pl.ds(r0, 16)]
            a0 = a0 + plsc.load_gather(ax[s], [ridx, jnp.full((16,), F, jnp.int32)])
            a1 = jnp.zeros((16,), jnp.float32)
            a2 = jnp.zeros((16,), jnp.float32)
            a3 = jnp.zeros((16,), jnp.float32)
            for f in range(F):
                cf = jnp.full((16,), f, jnp.int32)
                g_prod = plsc.load_gather(cu[s], [ridx, cf]) * plsc.load_gather(ci[s], [ridx, cbi + f])
                t_prod = plsc.load_gather(cu[s], [ridx, cf + F]) * plsc.load_gather(ax[s], [ridx, cf])
                if f % 2 == 0:
                    a0 = a0 + g_prod
                    a2 = a2 + t_prod
                else:
                    a1 = a1 + g_prod
                    a3 = a3 + t_prod
            ob[s][pl.ds(r0, 16)] = (a0 + a1) + (a2 + a3)
            return carry

        lax.fori_loop(0, _CHUNK // 16, group, 0)
        base = pl.multiple_of(base_w + ch * _CHUNK, _CHUNK)
        pltpu.sync_copy(ob[s], out_h.at[pl.ds(base, _CHUNK)])


def _sc_combine(user, item, aux, bu_w, bi_w, cat_u, gi2):
    mesh = plsc.VectorSubcoreMesh(core_axis_name="c", subcore_axis_name="s")
    dbuf = lambda shape, dt: [pltpu.VMEM(shape, dt), pltpu.VMEM(shape, dt)]
    return pl.kernel(
        _sc_body,
        out_type=jax.ShapeDtypeStruct((B,), jnp.float32),
        mesh=mesh,
        compiler_params=pltpu.CompilerParams(
            needs_layout_passes=False, use_tc_tiling_on_sc=False),
        scratch_types=[
            pltpu.VMEM((_NCHUNK, _CHUNK), jnp.int32),
            pltpu.VMEM((_NCHUNK, _CHUNK), jnp.int32),
            pltpu.VMEM((_NCHUNK, _CHUNK), jnp.int32),
            *dbuf((_CHUNK, C), jnp.float32),
            *dbuf((_CHUNK, C), jnp.float32),
            *dbuf((_CHUNK, C), jnp.float32),
            *dbuf((_CHUNK,), jnp.float32),
            *dbuf((_CHUNK,), jnp.float32),
            *dbuf((_CHUNK,), jnp.float32),
            pltpu.SemaphoreType.DMA,
            pltpu.SemaphoreType.DMA,
            pltpu.SemaphoreType.DMA,
        ],
    )(user, item, aux, bu_w, bi_w, cat_u, gi2)


def kernel(user, item_i, cnn_feature_i, alpha, beta_u_w, beta_i_w,
           gamma_u_w, gamma_i_w, theta_u_w, E_w, beta_p_w):
    user = user.astype(jnp.int32)
    item = item_i.astype(jnp.int32)
    W = jnp.concatenate(
        [E_w, beta_p_w, jnp.zeros((C - F - 1, C), jnp.float32)], axis=0)
    aux = _tc_dense(cnn_feature_i, W, alpha)
    cat_u = jnp.concatenate([gamma_u_w, theta_u_w], axis=1)
    I2 = gamma_i_w.shape[0] // 2
    out = _sc_combine(user, item, aux,
                      beta_u_w.reshape(-1), beta_i_w.reshape(-1),
                      cat_u, gamma_i_w.reshape(I2, C))
    return out.reshape(1, B)


# speedup vs baseline: 1.5861x; 1.2594x over previous
import jax
import jax.numpy as jnp
from jax import lax
from jax.experimental import pallas as pl
from jax.experimental.pallas import tpu as pltpu
from jax.experimental.pallas import tpu_sc as plsc

B = 16384
F = 64
C = 128

_NC = 2
_NS = 16
_NW = _NC * _NS
_ROWS_PER_W = B // _NW
_CHUNK = 128
_NCHUNK = _ROWS_PER_W // _CHUNK


def _tc_body(cnn_ref, w_ref, alpha_ref, aux_ref):
    cnn = cnn_ref[...]
    aux = lax.dot_general(cnn, w_ref[...], (((1,), (1,)), ((), ())),
                          preferred_element_type=jnp.float32)
    is_dense_col = (lax.broadcasted_iota(jnp.int32, (1, C), 1) == F)
    aux_ref[...] = aux + jnp.where(is_dense_col, alpha_ref[0, 0], 0.0)


def _tc_dense(cnn, W, alpha):
    return pl.pallas_call(
        _tc_body,
        out_shape=jax.ShapeDtypeStruct((B, C), jnp.float32),
        in_specs=[
            pl.BlockSpec(memory_space=pltpu.MemorySpace.VMEM),
            pl.BlockSpec(memory_space=pltpu.MemorySpace.VMEM),
            pl.BlockSpec(memory_space=pltpu.MemorySpace.SMEM),
        ],
        out_specs=pl.BlockSpec(memory_space=pltpu.MemorySpace.VMEM),
    )(cnn, W, alpha)


def _sc_body(user_h, item_h, aux_h, bu_h, bi_h, cat_h, gi_h,
             out_h, idx_u2, idx_i2, idx_ih2,
             cu0, cu1, ci0, ci1, ax0, ax1, bu0, bu1, bi0, bi1, ob0, ob1,
             sem_idx, sem0, sem1):
    wid = lax.axis_index("s") * _NC + lax.axis_index("c")
    base_w = wid * _ROWS_PER_W
    riota = lax.iota(jnp.int32, 16)

    cu = (cu0, cu1)
    ci = (ci0, ci1)
    ax = (ax0, ax1)
    bu = (bu0, bu1)
    bi = (bi0, bi1)
    ob = (ob0, ob1)
    sems = (sem0, sem1)

    idx_cps = []
    for ch in range(_NCHUNK):
        base = pl.multiple_of(base_w + ch * _CHUNK, _CHUNK)
        idx_cps.append(
            pltpu.async_copy(user_h.at[pl.ds(base, _CHUNK)], idx_u2.at[ch],
                             sem_idx))
        idx_cps.append(
            pltpu.async_copy(item_h.at[pl.ds(base, _CHUNK)], idx_i2.at[ch],
                             sem_idx))
    for cp in idx_cps:
        cp.wait()
    for ch in range(_NCHUNK):
        for j in range(_CHUNK // 16):
            idx_ih2[ch, pl.ds(j * 16, 16)] = idx_i2[ch, pl.ds(j * 16, 16)] >> 1

    def issue(ch, s):
        base = pl.multiple_of(base_w + ch * _CHUNK, _CHUNK)
        return (
            pltpu.async_copy(cat_h.at[idx_u2.at[ch]], cu[s], sems[s]),
            pltpu.async_copy(gi_h.at[idx_ih2.at[ch]], ci[s], sems[s]),
            pltpu.async_copy(aux_h.at[pl.ds(base, _CHUNK), :], ax[s], sems[s]),
            pltpu.async_copy(bu_h.at[idx_u2.at[ch]], bu[s], sems[s]),
            pltpu.async_copy(bi_h.at[idx_i2.at[ch]], bi[s], sems[s]),
        )

    inflight = issue(0, 0)
    for ch in range(_NCHUNK):
        s = ch % 2
        nxt = None
        if ch + 1 < _NCHUNK:
            nxt = issue(ch + 1, (ch + 1) % 2)
        for cp in inflight:
            cp.wait()
        inflight = nxt

        def group(g, carry):
            r0 = pl.multiple_of(g * 16, 16)
            ridx = riota + g * 16
            cbi16 = (idx_i2[ch, pl.ds(r0, 16)] & 1) << 6
            base16 = bu[s][pl.ds(r0, 16)] + bi[s][pl.ds(r0, 16)]
            base16 = base16 + plsc.load_gather(
                ax[s], [ridx, jnp.full((16,), F, jnp.int32)])
            lane0 = riota == 0
            for r in range(16):
                rr = r0 + r
                cbi = cbi16[r]
                pa = jnp.zeros((16,), jnp.float32)
                pb = jnp.zeros((16,), jnp.float32)
                for j in range(F // 16):
                    gu_v = cu[s][rr, pl.ds(j * 16, 16)]
                    tu_v = cu[s][rr, pl.ds(F + j * 16, 16)]
                    gi_v = ci[s][rr, pl.ds(cbi + j * 16, 16)]
                    pj_v = ax[s][rr, pl.ds(j * 16, 16)]
                    pa = pa + gu_v * gi_v
                    pb = pb + tu_v * pj_v
                tot = jnp.sum(pa + pb) + base16[r]
                plsc.store_scatter(ob[s], [jnp.full((16,), rr, jnp.int32)],
                                   jnp.full((16,), tot, jnp.float32),
                                   mask=lane0)
            return carry

        lax.fori_loop(0, _CHUNK // 16, group, 0)
        base = pl.multiple_of(base_w + ch * _CHUNK, _CHUNK)
        pltpu.sync_copy(ob[s], out_h.at[pl.ds(base, _CHUNK)])


def _sc_combine(user, item, aux, bu_w, bi_w, cat_u, gi2):
    mesh = plsc.VectorSubcoreMesh(core_axis_name="c", subcore_axis_name="s")
    dbuf = lambda shape, dt: [pltpu.VMEM(shape, dt), pltpu.VMEM(shape, dt)]
    return pl.kernel(
        _sc_body,
        out_type=jax.ShapeDtypeStruct((B,), jnp.float32),
        mesh=mesh,
        compiler_params=pltpu.CompilerParams(
            needs_layout_passes=False, use_tc_tiling_on_sc=False),
        scratch_types=[
            pltpu.VMEM((_NCHUNK, _CHUNK), jnp.int32),
            pltpu.VMEM((_NCHUNK, _CHUNK), jnp.int32),
            pltpu.VMEM((_NCHUNK, _CHUNK), jnp.int32),
            *dbuf((_CHUNK, C), jnp.float32),
            *dbuf((_CHUNK, C), jnp.float32),
            *dbuf((_CHUNK, C), jnp.float32),
            *dbuf((_CHUNK,), jnp.float32),
            *dbuf((_CHUNK,), jnp.float32),
            *dbuf((_CHUNK,), jnp.float32),
            pltpu.SemaphoreType.DMA,
            pltpu.SemaphoreType.DMA,
            pltpu.SemaphoreType.DMA,
        ],
    )(user, item, aux, bu_w, bi_w, cat_u, gi2)


def kernel(user, item_i, cnn_feature_i, alpha, beta_u_w, beta_i_w,
           gamma_u_w, gamma_i_w, theta_u_w, E_w, beta_p_w):
    user = user.astype(jnp.int32)
    item = item_i.astype(jnp.int32)
    W = jnp.concatenate(
        [E_w, beta_p_w, jnp.zeros((C - F - 1, C), jnp.float32)], axis=0)
    aux = _tc_dense(cnn_feature_i, W, alpha)
    cat_u = jnp.concatenate([gamma_u_w, theta_u_w], axis=1)
    I2 = gamma_i_w.shape[0] // 2
    out = _sc_combine(user, item, aux,
                      beta_u_w.reshape(-1), beta_i_w.reshape(-1),
                      cat_u, gamma_i_w.reshape(I2, C))
    return out.reshape(1, B)
